# baseline (speedup 1.0000x reference)
import jax
import jax.numpy as jnp
from jax import lax
from jax.experimental import pallas as pl
from jax.experimental.pallas import tpu as pltpu

_T = 64
_B = 64
_D = 256
_H = 512
_TC = 8
_NC = _T // _TC
_ROWS = _TC * _B
_NG = _NC // 2


def _gru_fused_kernel(tok_ref, emb_ref, h0_ref, wih_ref, whh_ref, bih_ref,
                      bhh_ref, out_ref, hT_ref,
                      xbuf0_ref, xbuf1_ref, sem0, sem1,
                      wih_bf_ref, whh_bf_ref, h_ref, gi_ref):
    g = pl.program_id(0)
    B, H, TC = _B, _H, _TC

    def issue(chunk, xbuf, sem):
        base = chunk * _ROWS
        for i in range(_ROWS):
            pltpu.make_async_copy(
                emb_ref.at[pl.ds(tok_ref[base + i], 1), :],
                xbuf.at[pl.ds(i, 1), :],
                sem).start()

    def drain(xbuf, sem):
        pltpu.make_async_copy(xbuf, xbuf, sem).wait()

    @pl.when(g == 0)
    def _prologue():
        issue(0, xbuf0_ref, sem0)
        wih_bf_ref[...] = wih_ref[...].astype(jnp.bfloat16)
        whh_bf_ref[...] = whh_ref[...].astype(jnp.bfloat16)
        h_ref[...] = h0_ref[...]

    bhh = bhh_ref[...]
    bias = bih_ref[...] + jnp.concatenate(
        [bhh[:, :2 * H], jnp.zeros((1, H), jnp.float32)], axis=1)
    b_hn = jnp.broadcast_to(bhh[:, 2 * H:], (B, H))
    bidx = lax.broadcasted_iota(jnp.int32, (B, 1), 0)

    def half(xbuf, sem, kbase):
        drain(xbuf, sem)
        xc = xbuf[...].astype(jnp.bfloat16)
        gi_ref[...] = (
            jnp.dot(xc, wih_bf_ref[...], preferred_element_type=jnp.float32)
            + bias)
        h = h_ref[...]
        for k in range(TC):
            t = (2 * g + kbase // TC) * TC + k
            gi = gi_ref[k * B:(k + 1) * B, :]
            gh = jnp.dot(h.astype(jnp.bfloat16), whh_bf_ref[...],
                         preferred_element_type=jnp.float32)
            r = jax.nn.sigmoid(gi[:, 0:H] + gh[:, 0:H])
            z = jax.nn.sigmoid(gi[:, H:2 * H] + gh[:, H:2 * H])
            n = jnp.tanh(gi[:, 2 * H:] + r * (gh[:, 2 * H:] + b_hn))
            h_new = (1.0 - z) * n + z * h
            mask = (bidx + t) < _T
            out_ref[:, kbase + k, :] = jnp.where(mask, h_new, 0.0)
            h = jnp.where(mask, h_new, h)
        h_ref[...] = h

    issue(2 * g + 1, xbuf1_ref, sem1)
    half(xbuf0_ref, sem0, 0)
    issue(lax.rem(2 * g + 2, _NC), xbuf0_ref, sem0)
    half(xbuf1_ref, sem1, TC)

    @pl.when(g == _NG - 1)
    def _fin():
        hT_ref[...] = h_ref[...]
        drain(xbuf0_ref, sem0)


def kernel(emb, w_ih_t, w_hh_t, b_ih, b_hh, tokens, hidden):
    T, B, D, H, TC = _T, _B, _D, _H, _TC

    tokens_flat = tokens.T.reshape(T * B)
    h0 = hidden[0]

    output, h_final = pl.pallas_call(
        _gru_fused_kernel,
        out_shape=(
            jax.ShapeDtypeStruct((B, T, H), jnp.float32),
            jax.ShapeDtypeStruct((B, H), jnp.float32),
        ),
        grid_spec=pltpu.PrefetchScalarGridSpec(
            num_scalar_prefetch=1,
            grid=(_NG,),
            in_specs=[
                pl.BlockSpec(memory_space=pltpu.MemorySpace.HBM),
                pl.BlockSpec((B, H), lambda t, *_: (0, 0)),
                pl.BlockSpec((D, 3 * H), lambda t, *_: (0, 0)),
                pl.BlockSpec((H, 3 * H), lambda t, *_: (0, 0)),
                pl.BlockSpec((1, 3 * H), lambda t, *_: (0, 0)),
                pl.BlockSpec((1, 3 * H), lambda t, *_: (0, 0)),
            ],
            out_specs=[
                pl.BlockSpec((B, 2 * TC, H), lambda t, *_: (0, t, 0)),
                pl.BlockSpec((B, H), lambda t, *_: (0, 0)),
            ],
            scratch_shapes=[
                pltpu.VMEM((_ROWS, D), jnp.float32),
                pltpu.VMEM((_ROWS, D), jnp.float32),
                pltpu.SemaphoreType.DMA,
                pltpu.SemaphoreType.DMA,
                pltpu.VMEM((D, 3 * H), jnp.bfloat16),
                pltpu.VMEM((H, 3 * H), jnp.bfloat16),
                pltpu.VMEM((B, H), jnp.float32),
                pltpu.VMEM((_ROWS, 3 * H), jnp.float32),
            ],
        ),
        compiler_params=pltpu.CompilerParams(
            dimension_semantics=("arbitrary",)),
    )(tokens_flat, emb, h0, w_ih_t, w_hh_t, b_ih, b_hh)

    return output, h_final[None]


# speedup vs baseline: 1.0411x; 1.0411x over previous
import jax
import jax.numpy as jnp
from jax import lax
from jax.experimental import pallas as pl
from jax.experimental.pallas import tpu as pltpu

_T = 64
_B = 64
_D = 256
_H = 512
_V = 20000
_TC = 8
_NC = _T // _TC
_ROWS = _TC * _B


def _gru_fused_kernel(tok_ref, emb_ref, h0_ref, wih_ref, whh_ref, bih_ref,
                      bhh_ref, out_ref, hT_ref,
                      xstage_ref, wih_bf_ref, whh_bf_ref, h_ref, gi_ref):
    tau = pl.program_id(0)
    B, H, TC = _B, _H, _TC

    @pl.when(tau == 0)
    def _init():
        wih_bf_ref[...] = wih_ref[...].astype(jnp.bfloat16)
        whh_bf_ref[...] = whh_ref[...].astype(jnp.bfloat16)
        h_ref[...] = h0_ref[...]

    bhh = bhh_ref[...]
    bias = bih_ref[...] + jnp.concatenate(
        [bhh[:, :2 * H], jnp.zeros((1, H), jnp.float32)], axis=1)
    b_hn = jnp.broadcast_to(bhh[:, 2 * H:], (B, H))
    bidx = lax.broadcasted_iota(jnp.int32, (B, 1), 0)

    base = tau * _ROWS
    for i in range(_ROWS):
        xstage_ref[pl.ds(i, 1), :] = (
            emb_ref[pl.ds(tok_ref[base + i], 1), :].astype(jnp.bfloat16))

    gi_ref[...] = (
        jnp.dot(xstage_ref[...], wih_bf_ref[...],
                preferred_element_type=jnp.float32)
        + bias)

    h = h_ref[...]
    for k in range(TC):
        t = tau * TC + k
        gi = gi_ref[k * B:(k + 1) * B, :]
        gh = jnp.dot(h.astype(jnp.bfloat16), whh_bf_ref[...],
                     preferred_element_type=jnp.float32)
        r = jax.nn.sigmoid(gi[:, 0:H] + gh[:, 0:H])
        z = jax.nn.sigmoid(gi[:, H:2 * H] + gh[:, H:2 * H])
        n = jnp.tanh(gi[:, 2 * H:] + r * (gh[:, 2 * H:] + b_hn))
        h_new = (1.0 - z) * n + z * h
        mask = (bidx + t) < _T
        out_ref[:, k, :] = jnp.where(mask, h_new, 0.0)
        h = jnp.where(mask, h_new, h)
    h_ref[...] = h

    @pl.when(tau == _NC - 1)
    def _fin():
        hT_ref[...] = h


def kernel(emb, w_ih_t, w_hh_t, b_ih, b_hh, tokens, hidden):
    T, B, D, H, TC = _T, _B, _D, _H, _TC

    tokens_flat = tokens.T.reshape(T * B)
    h0 = hidden[0]

    output, h_final = pl.pallas_call(
        _gru_fused_kernel,
        out_shape=(
            jax.ShapeDtypeStruct((B, T, H), jnp.float32),
            jax.ShapeDtypeStruct((B, H), jnp.float32),
        ),
        grid_spec=pltpu.PrefetchScalarGridSpec(
            num_scalar_prefetch=1,
            grid=(_NC,),
            in_specs=[
                pl.BlockSpec((_V, D), lambda t, *_: (0, 0)),
                pl.BlockSpec((B, H), lambda t, *_: (0, 0)),
                pl.BlockSpec((D, 3 * H), lambda t, *_: (0, 0)),
                pl.BlockSpec((H, 3 * H), lambda t, *_: (0, 0)),
                pl.BlockSpec((1, 3 * H), lambda t, *_: (0, 0)),
                pl.BlockSpec((1, 3 * H), lambda t, *_: (0, 0)),
            ],
            out_specs=[
                pl.BlockSpec((B, TC, H), lambda t, *_: (0, t, 0)),
                pl.BlockSpec((B, H), lambda t, *_: (0, 0)),
            ],
            scratch_shapes=[
                pltpu.VMEM((_ROWS, D), jnp.bfloat16),
                pltpu.VMEM((D, 3 * H), jnp.bfloat16),
                pltpu.VMEM((H, 3 * H), jnp.bfloat16),
                pltpu.VMEM((B, H), jnp.float32),
                pltpu.VMEM((_ROWS, 3 * H), jnp.float32),
            ],
        ),
        compiler_params=pltpu.CompilerParams(
            dimension_semantics=("arbitrary",),
            vmem_limit_bytes=100 * 1024 * 1024),
    )(tokens_flat, emb, h0, w_ih_t, w_hh_t, b_ih, b_hh)

    return output, h_final[None]
